# baseline (speedup 1.0000x reference)
import jax
import jax.numpy as jnp
from jax.experimental import pallas as pl
from jax.experimental.pallas import tpu as pltpu
from jax.experimental.pallas import tpu_sc as plsc

_CODEBOOK = 1024
_DIM = 64
_COMMIT = 0.25
_GATHER_WINDOW = 128
_B = 32
_T = 1024
_CHUNK = 16


def _idx_body(x_ref, emb_ref, x2_ref, e2_ref, idx_ref):
    xb = x_ref[0]
    emb = emb_ref[...]
    x2 = x2_ref[0]
    e2 = e2_ref[...]
    xe = jax.lax.dot_general(
        emb, xb, (((1,), (0,)), ((), ())),
        preferred_element_type=jnp.float32)
    d = (x2 + e2) - 2.0 * xe
    dmin = jnp.min(d, axis=0)
    iota = jax.lax.broadcasted_iota(jnp.int32, d.shape, 0)
    idx_ref[0, 0] = jnp.min(jnp.where(d == dmin[None, :], iota, _CODEBOOK),
                            axis=0)


def _idx_chunk(x3, emb_weight, x2, e2, off):
    return pl.pallas_call(
        _idx_body,
        grid=(_CHUNK,),
        in_specs=[
            pl.BlockSpec((1, _DIM, _T), lambda b: (b + off, 0, 0)),
            pl.BlockSpec((_CODEBOOK, _DIM), lambda b: (0, 0)),
            pl.BlockSpec((1, 1, _T), lambda b: (b + off, 0, 0)),
            pl.BlockSpec((_CODEBOOK, 1), lambda b: (0, 0)),
        ],
        out_specs=pl.BlockSpec((1, 1, _T), lambda b: (b, 0, 0)),
        out_shape=jax.ShapeDtypeStruct((_CHUNK, 1, _T), jnp.int32),
    )(x3, emb_weight, x2, e2)


def _sc_gather(emb_pad, idx_flat, n_rows):
    mesh = plsc.VectorSubcoreMesh(core_axis_name="c", subcore_axis_name="s")

    @pl.kernel(out_type=jax.ShapeDtypeStruct((n_rows, 128), jnp.float32),
               mesh=mesh)
    def gather_kernel(emb_hbm, i_hbm, o_hbm):
        def body(i_vmem, o_vmem):
            pltpu.sync_copy(emb_hbm.at[i_vmem.at[0]], o_vmem)

        pltpu.emit_pipeline(
            body,
            grid=(n_rows // _GATHER_WINDOW,),
            in_specs=[pl.BlockSpec((1, _GATHER_WINDOW),
                                   index_map=lambda i: (0, i))],
            out_specs=[pl.BlockSpec((_GATHER_WINDOW, 128),
                                    index_map=lambda i: (i, 0))],
            core_axis_name=("c", "s"),
            dimension_semantics=(pltpu.PARALLEL,),
        )(i_hbm, o_hbm)

    return gather_kernel(emb_pad, idx_flat)


def _st_body(x_ref, q_ref, out_ref, loss_ref):
    b = pl.program_id(0)
    xb = x_ref[0]
    qT = jnp.transpose(q_ref[0][:, :_DIM])
    out_ref[0] = xb + (qT - xb)
    part = jnp.sum((qT - xb) ** 2)

    @pl.when(b == 0)
    def _():
        loss_ref[0, 0] = 0.0

    loss_ref[0, 0] += part


def _st_body_alias(x_ref, q_ref, prev_ref, out_ref, loss_ref):
    _st_body(x_ref, q_ref, out_ref, loss_ref)


def _st_chunk_first(x3, q_chunk, off):
    return pl.pallas_call(
        _st_body,
        grid=(_CHUNK,),
        in_specs=[
            pl.BlockSpec((1, _DIM, _T), lambda b: (b + off, 0, 0)),
            pl.BlockSpec((1, _T, 128), lambda b: (b, 0, 0)),
        ],
        out_specs=[
            pl.BlockSpec((1, _DIM, _T), lambda b: (b + off, 0, 0)),
            pl.BlockSpec(block_shape=(1, 1), index_map=lambda b: (0, 0),
                         memory_space=pltpu.MemorySpace.SMEM),
        ],
        out_shape=[
            jax.ShapeDtypeStruct((_B, _DIM, _T), jnp.float32),
            jax.ShapeDtypeStruct((1, 1), jnp.float32),
        ],
    )(x3, q_chunk)


def _st_chunk_alias(x3, q_chunk, prev, off):
    return pl.pallas_call(
        _st_body_alias,
        grid=(_CHUNK,),
        in_specs=[
            pl.BlockSpec((1, _DIM, _T), lambda b: (b + off, 0, 0)),
            pl.BlockSpec((1, _T, 128), lambda b: (b, 0, 0)),
            pl.BlockSpec(memory_space=pltpu.MemorySpace.HBM),
        ],
        out_specs=[
            pl.BlockSpec((1, _DIM, _T), lambda b: (b + off, 0, 0)),
            pl.BlockSpec(block_shape=(1, 1), index_map=lambda b: (0, 0),
                         memory_space=pltpu.MemorySpace.SMEM),
        ],
        out_shape=[
            jax.ShapeDtypeStruct((_B, _DIM, _T), jnp.float32),
            jax.ShapeDtypeStruct((1, 1), jnp.float32),
        ],
        input_output_aliases={2: 0},
    )(x3, q_chunk, prev)


def kernel(x, emb_weight):
    B, C, H, W = x.shape
    T = H * W
    x3 = x.reshape(B, C, T)
    flat_x = jnp.transpose(x, (0, 2, 3, 1)).reshape(-1, C)
    x2 = jnp.sum(flat_x ** 2, axis=1).reshape(B, 1, T)
    e2 = jnp.sum(emb_weight ** 2, axis=1).reshape(_CODEBOOK, 1)
    emb_pad = jnp.concatenate(
        [emb_weight, jnp.zeros((_CODEBOOK, 128 - _DIM), jnp.float32)], axis=1)

    idx_a = _idx_chunk(x3, emb_weight, x2, e2, 0)
    idx_b = _idx_chunk(x3, emb_weight, x2, e2, _CHUNK)
    q_a = _sc_gather(emb_pad, idx_a.reshape(1, _CHUNK * T), _CHUNK * T)
    q_b = _sc_gather(emb_pad, idx_b.reshape(1, _CHUNK * T), _CHUNK * T)

    out_a, loss_a = _st_chunk_first(x3, q_a.reshape(_CHUNK, T, 128), 0)
    q3, loss_b = _st_chunk_alias(x3, q_b.reshape(_CHUNK, T, 128), out_a,
                                 _CHUNK)

    m = (loss_a[0, 0] + loss_b[0, 0]) / (B * C * H * W)
    loss = m + _COMMIT * m
    return q3.reshape(B, C, H, W), loss


# speedup vs baseline: 1.0030x; 1.0030x over previous
import jax
import jax.numpy as jnp
from jax.experimental import pallas as pl
from jax.experimental.pallas import tpu as pltpu
from jax.experimental.pallas import tpu_sc as plsc

_CODEBOOK = 1024
_DIM = 64
_COMMIT = 0.25
_GATHER_WINDOW = 128
_B = 32
_T = 1024
_CHUNK = 16


def _idx_body(x_ref, emb_ref, x2_ref, e2_ref, idx_ref):
    xb = x_ref[0]
    emb = emb_ref[...]
    x2 = x2_ref[0]
    e2 = e2_ref[...]
    xe2 = jax.lax.dot_general(
        emb, xb, (((1,), (0,)), ((), ())),
        preferred_element_type=jnp.float32)
    d = (x2 + e2) - xe2
    dmin = jnp.min(d, axis=0)
    iota = jax.lax.broadcasted_iota(jnp.int32, d.shape, 0)
    idx_ref[0, 0] = jnp.min(jnp.where(d == dmin[None, :], iota, _CODEBOOK),
                            axis=0)


def _idx_chunk(x3, emb2, x2, e2, off):
    return pl.pallas_call(
        _idx_body,
        grid=(_CHUNK,),
        in_specs=[
            pl.BlockSpec((1, _DIM, _T), lambda b: (b + off, 0, 0)),
            pl.BlockSpec((_CODEBOOK, _DIM), lambda b: (0, 0)),
            pl.BlockSpec((1, 1, _T), lambda b: (b + off, 0, 0)),
            pl.BlockSpec((_CODEBOOK, 1), lambda b: (0, 0)),
        ],
        out_specs=pl.BlockSpec((1, 1, _T), lambda b: (b, 0, 0)),
        out_shape=jax.ShapeDtypeStruct((_CHUNK, 1, _T), jnp.int32),
    )(x3, emb2, x2, e2)


def _sc_gather(emb_pad, idx_flat, n_rows):
    mesh = plsc.VectorSubcoreMesh(core_axis_name="c", subcore_axis_name="s")

    @pl.kernel(out_type=jax.ShapeDtypeStruct((n_rows, 128), jnp.float32),
               mesh=mesh)
    def gather_kernel(emb_hbm, i_hbm, o_hbm):
        def body(i_vmem, o_vmem):
            pltpu.sync_copy(emb_hbm.at[i_vmem.at[0]], o_vmem)

        pltpu.emit_pipeline(
            body,
            grid=(n_rows // _GATHER_WINDOW,),
            in_specs=[pl.BlockSpec((1, _GATHER_WINDOW),
                                   index_map=lambda i: (0, i))],
            out_specs=[pl.BlockSpec((_GATHER_WINDOW, 128),
                                    index_map=lambda i: (i, 0))],
            core_axis_name=("c", "s"),
            dimension_semantics=(pltpu.PARALLEL,),
        )(i_hbm, o_hbm)

    return gather_kernel(emb_pad, idx_flat)


def _st_body(x_ref, q_ref, out_ref, loss_ref):
    b = pl.program_id(0)
    xb = x_ref[0]
    qT = jnp.transpose(q_ref[0][:, :_DIM])
    out_ref[0] = xb + (qT - xb)
    part = jnp.sum((qT - xb) ** 2)

    @pl.when(b == 0)
    def _():
        loss_ref[0, 0] = 0.0

    loss_ref[0, 0] += part


def _st_body_alias(x_ref, q_ref, prev_ref, out_ref, loss_ref):
    _st_body(x_ref, q_ref, out_ref, loss_ref)


def _st_chunk_first(x3, q_chunk, off):
    return pl.pallas_call(
        _st_body,
        grid=(_CHUNK,),
        in_specs=[
            pl.BlockSpec((1, _DIM, _T), lambda b: (b + off, 0, 0)),
            pl.BlockSpec((1, _T, 128), lambda b: (b, 0, 0)),
        ],
        out_specs=[
            pl.BlockSpec((1, _DIM, _T), lambda b: (b + off, 0, 0)),
            pl.BlockSpec(block_shape=(1, 1), index_map=lambda b: (0, 0),
                         memory_space=pltpu.MemorySpace.SMEM),
        ],
        out_shape=[
            jax.ShapeDtypeStruct((_B, _DIM, _T), jnp.float32),
            jax.ShapeDtypeStruct((1, 1), jnp.float32),
        ],
    )(x3, q_chunk)


def _st_chunk_alias(x3, q_chunk, prev, off):
    return pl.pallas_call(
        _st_body_alias,
        grid=(_CHUNK,),
        in_specs=[
            pl.BlockSpec((1, _DIM, _T), lambda b: (b + off, 0, 0)),
            pl.BlockSpec((1, _T, 128), lambda b: (b, 0, 0)),
            pl.BlockSpec(memory_space=pltpu.MemorySpace.HBM),
        ],
        out_specs=[
            pl.BlockSpec((1, _DIM, _T), lambda b: (b + off, 0, 0)),
            pl.BlockSpec(block_shape=(1, 1), index_map=lambda b: (0, 0),
                         memory_space=pltpu.MemorySpace.SMEM),
        ],
        out_shape=[
            jax.ShapeDtypeStruct((_B, _DIM, _T), jnp.float32),
            jax.ShapeDtypeStruct((1, 1), jnp.float32),
        ],
        input_output_aliases={2: 0},
    )(x3, q_chunk, prev)


def kernel(x, emb_weight):
    B, C, H, W = x.shape
    T = H * W
    x3 = x.reshape(B, C, T)
    flat_x = jnp.transpose(x, (0, 2, 3, 1)).reshape(-1, C)
    x2 = jnp.sum(flat_x ** 2, axis=1).reshape(B, 1, T)
    e2 = jnp.sum(emb_weight ** 2, axis=1).reshape(_CODEBOOK, 1)
    emb_x2 = emb_weight * 2.0
    emb_pad = jnp.concatenate(
        [emb_weight, jnp.zeros((_CODEBOOK, 128 - _DIM), jnp.float32)], axis=1)

    idx_a = _idx_chunk(x3, emb_x2, x2, e2, 0)
    idx_b = _idx_chunk(x3, emb_x2, x2, e2, _CHUNK)
    q_a = _sc_gather(emb_pad, idx_a.reshape(1, _CHUNK * T), _CHUNK * T)
    q_b = _sc_gather(emb_pad, idx_b.reshape(1, _CHUNK * T), _CHUNK * T)

    out_a, loss_a = _st_chunk_first(x3, q_a.reshape(_CHUNK, T, 128), 0)
    q3, loss_b = _st_chunk_alias(x3, q_b.reshape(_CHUNK, T, 128), out_a,
                                 _CHUNK)

    m = (loss_a[0, 0] + loss_b[0, 0]) / (B * C * H * W)
    loss = m + _COMMIT * m
    return q3.reshape(B, C, H, W), loss
